# baseline (speedup 1.0000x reference)
import functools

import jax
import jax.numpy as jnp
from jax import lax
from jax.experimental import pallas as pl
from jax.experimental.pallas import tpu as pltpu
from jax.experimental.pallas import tpu_sc as plsc

EPS = 1e-5
_CHUNK = 64
_NBUF = 2


def _make_sc_gather(V, D, B, NC, NW):
    b_per_w = B // NW
    mesh = plsc.VectorSubcoreMesh(core_axis_name="c", subcore_axis_name="s")

    @functools.partial(
        pl.kernel,
        mesh=mesh,
        out_type=jax.ShapeDtypeStruct((B, D), jnp.float32),
        scratch_types=[
            pltpu.VMEM((b_per_w,), jnp.int32),
            pltpu.VMEM((b_per_w, D), jnp.float32),
            pltpu.SemaphoreType.DMA,
        ],
        compiler_params=pltpu.CompilerParams(
            needs_layout_passes=False, use_tc_tiling_on_sc=True
        ),
    )
    def gather(table_hbm, idx_hbm, out_hbm, idx_v, rows_v, sem):
        wid = lax.axis_index("s") * NC + lax.axis_index("c")
        base = wid * b_per_w
        pltpu.sync_copy(idx_hbm.at[wid], idx_v)
        lane = lax.iota(jnp.int32, 16)

        def chunk16(i):
            tv = idx_v[pl.ds(i * 16, 16)]
            for j in range(16):
                r = jnp.sum(jnp.where(lane == j, tv, 0))
                pltpu.async_copy(
                    table_hbm.at[pl.ds(r, 1)],
                    rows_v.at[pl.ds(i * 16 + j, 1)],
                    sem,
                )

        pl.loop(0, b_per_w // 16)(chunk16)
        pltpu.make_async_copy(
            table_hbm.at[pl.ds(0, b_per_w)], rows_v, sem
        ).wait()
        pltpu.sync_copy(rows_v, out_hbm.at[pl.ds(base, b_per_w)])

    return gather


def _dense_body(x_ref, w_ref, b_ref, g_ref, bt_ref, o_ref):
    x = x_ref[...]
    h = jnp.dot(x, w_ref[...], preferred_element_type=jnp.float32) + b_ref[...]
    h = jnp.maximum(h, 0.0)
    m = jnp.mean(h, axis=1, keepdims=True)
    c = h - m
    v = jnp.mean(c * c, axis=1, keepdims=True)
    o_ref[...] = c * lax.rsqrt(v + EPS) * g_ref[...] + bt_ref[...]


def _dense(rows, W, b, gamma, beta, BB=2048):
    B, D = rows.shape
    H = W.shape[1]
    return pl.pallas_call(
        _dense_body,
        grid=(B // BB,),
        in_specs=[
            pl.BlockSpec((BB, D), lambda i: (i, 0)),
            pl.BlockSpec((D, H), lambda i: (0, 0)),
            pl.BlockSpec((1, H), lambda i: (0, 0)),
            pl.BlockSpec((1, H), lambda i: (0, 0)),
            pl.BlockSpec((1, H), lambda i: (0, 0)),
        ],
        out_specs=pl.BlockSpec((BB, H), lambda i: (i, 0)),
        out_shape=jax.ShapeDtypeStruct((B, H), jnp.float32),
    )(rows, W, b.reshape(1, H), gamma.reshape(1, H), beta.reshape(1, H))


def kernel(user_input, table, W, b, gamma, beta):
    B = user_input.shape[0]
    V, D = table.shape
    info = plsc.get_sparse_core_info()
    NC, NS = info.num_cores, info.num_subcores
    NW = NC * NS
    idx = user_input.astype(jnp.int32).reshape(NW, B // NW)
    rows = _make_sc_gather(V, D, B, NC, NW)(table, idx)
    return _dense(rows, W, b, gamma, beta)


# speedup vs baseline: 1.1229x; 1.1229x over previous
import functools

import jax
import jax.numpy as jnp
from jax import lax
from jax.experimental import pallas as pl
from jax.experimental.pallas import tpu as pltpu
from jax.experimental.pallas import tpu_sc as plsc

EPS = 1e-5
_TBL = 6144


def _transpose_body(xt_ref, o_ref):
    x = xt_ref[...]
    eye = jnp.asarray(
        lax.broadcasted_iota(jnp.int32, (x.shape[0], x.shape[0]), 0)
        == lax.broadcasted_iota(jnp.int32, (x.shape[0], x.shape[0]), 1),
        jnp.float32,
    )
    o_ref[...] = lax.dot_general(
        x, eye, (((0,), (0,)), ((), ())), preferred_element_type=jnp.float32
    )


def _transpose(tblT):
    D, V = tblT.shape
    steps = (V + _TBL - 1) // _TBL
    return pl.pallas_call(
        _transpose_body,
        grid=(steps,),
        in_specs=[pl.BlockSpec((D, _TBL), lambda i: (0, i))],
        out_specs=pl.BlockSpec((_TBL, D), lambda i: (i, 0)),
        out_shape=jax.ShapeDtypeStruct((V, D), jnp.float32),
    )(tblT)


def _make_sc_gather(V, D, B, NC, NW):
    b_per_w = B // NW
    mesh = plsc.VectorSubcoreMesh(core_axis_name="c", subcore_axis_name="s")

    @functools.partial(
        pl.kernel,
        mesh=mesh,
        out_type=jax.ShapeDtypeStruct((B, D), jnp.float32),
        scratch_types=[
            pltpu.VMEM((b_per_w,), jnp.int32),
            pltpu.VMEM((b_per_w, D), jnp.float32),
            pltpu.SemaphoreType.DMA,
        ],
        compiler_params=pltpu.CompilerParams(needs_layout_passes=False),
    )
    def gather(table_hbm, idx_hbm, out_hbm, idx_v, rows_v, sem):
        wid = lax.axis_index("s") * NC + lax.axis_index("c")
        base = wid * b_per_w
        pltpu.sync_copy(idx_hbm.at[wid], idx_v)
        lane = lax.iota(jnp.int32, 16)

        def chunk16(i):
            tv = idx_v[pl.ds(i * 16, 16)]
            for j in range(16):
                r = jnp.sum(jnp.where(lane == j, tv, 0))
                pltpu.async_copy(
                    table_hbm.at[pl.ds(r, 1)],
                    rows_v.at[pl.ds(i * 16 + j, 1)],
                    sem,
                )

        pl.loop(0, b_per_w // 16)(chunk16)
        pltpu.make_async_copy(
            table_hbm.at[pl.ds(0, b_per_w)], rows_v, sem
        ).wait()
        pltpu.sync_copy(rows_v, out_hbm.at[pl.ds(base, b_per_w)])

    return gather


def _dense_body(x_ref, w_ref, b_ref, g_ref, bt_ref, o_ref):
    x = x_ref[...]
    h = jnp.dot(x, w_ref[...], preferred_element_type=jnp.float32) + b_ref[...]
    h = jnp.maximum(h, 0.0)
    m = jnp.mean(h, axis=1, keepdims=True)
    c = h - m
    v = jnp.mean(c * c, axis=1, keepdims=True)
    o_ref[...] = c * lax.rsqrt(v + EPS) * g_ref[...] + bt_ref[...]


def _dense(rows, W, b, gamma, beta, BB=2048):
    B, D = rows.shape
    H = W.shape[1]
    return pl.pallas_call(
        _dense_body,
        grid=(B // BB,),
        in_specs=[
            pl.BlockSpec((BB, D), lambda i: (i, 0)),
            pl.BlockSpec((D, H), lambda i: (0, 0)),
            pl.BlockSpec((1, H), lambda i: (0, 0)),
            pl.BlockSpec((1, H), lambda i: (0, 0)),
            pl.BlockSpec((1, H), lambda i: (0, 0)),
        ],
        out_specs=pl.BlockSpec((BB, H), lambda i: (i, 0)),
        out_shape=jax.ShapeDtypeStruct((B, H), jnp.float32),
    )(rows, W, b.reshape(1, H), gamma.reshape(1, H), beta.reshape(1, H))


def kernel(user_input, table, W, b, gamma, beta):
    B = user_input.shape[0]
    V, D = table.shape
    info = plsc.get_sparse_core_info()
    NC, NS = info.num_cores, info.num_subcores
    NW = NC * NS
    idx = user_input.astype(jnp.int32).reshape(NW, B // NW)
    table_rm = _transpose(table.T)
    rows = _make_sc_gather(V, D, B, NC, NW)(table_rm, idx)
    return _dense(rows, W, b, gamma, beta)


# speedup vs baseline: 1.3575x; 1.2089x over previous
import functools

import jax
import jax.numpy as jnp
from jax import lax
from jax.experimental import pallas as pl
from jax.experimental.pallas import tpu as pltpu
from jax.experimental.pallas import tpu_sc as plsc

EPS = 1e-5
_TBL = 24576


def _transpose_body(xt_ref, o_ref):
    x = xt_ref[...]
    eye = jnp.asarray(
        lax.broadcasted_iota(jnp.int32, (x.shape[0], x.shape[0]), 0)
        == lax.broadcasted_iota(jnp.int32, (x.shape[0], x.shape[0]), 1),
        jnp.float32,
    )
    o_ref[...] = lax.dot_general(
        x, eye, (((0,), (0,)), ((), ())), preferred_element_type=jnp.float32
    )


def _transpose(tblT):
    D, V = tblT.shape
    steps = (V + _TBL - 1) // _TBL
    return pl.pallas_call(
        _transpose_body,
        grid=(steps,),
        in_specs=[pl.BlockSpec((D, _TBL), lambda i: (0, i))],
        out_specs=pl.BlockSpec((_TBL, D), lambda i: (i, 0)),
        out_shape=jax.ShapeDtypeStruct((V, D), jnp.float32),
    )(tblT)


def _make_sc_gather(V, D, B, NC, NW):
    b_per_w = B // NW
    mesh = plsc.VectorSubcoreMesh(core_axis_name="c", subcore_axis_name="s")

    @functools.partial(
        pl.kernel,
        mesh=mesh,
        out_type=jax.ShapeDtypeStruct((B, D), jnp.float32),
        scratch_types=[
            pltpu.VMEM((b_per_w,), jnp.int32),
            pltpu.VMEM((b_per_w, D), jnp.float32),
            pltpu.SemaphoreType.DMA,
        ],
        compiler_params=pltpu.CompilerParams(needs_layout_passes=False),
    )
    def gather(table_hbm, idx_hbm, out_hbm, idx_v, rows_v, sem):
        wid = lax.axis_index("s") * NC + lax.axis_index("c")
        base = wid * b_per_w
        pltpu.sync_copy(idx_hbm.at[wid], idx_v)
        lane = lax.iota(jnp.int32, 16)

        def chunk16(i):
            tv = idx_v[pl.ds(i * 16, 16)]
            for j in range(16):
                r = jnp.sum(jnp.where(lane == j, tv, 0))
                pltpu.async_copy(
                    table_hbm.at[pl.ds(r, 1)],
                    rows_v.at[pl.ds(i * 16 + j, 1)],
                    sem,
                )

        pl.loop(0, b_per_w // 16)(chunk16)
        pltpu.make_async_copy(
            table_hbm.at[pl.ds(0, b_per_w)], rows_v, sem
        ).wait()
        pltpu.sync_copy(rows_v, out_hbm.at[pl.ds(base, b_per_w)])

    return gather


def _dense_body(x_ref, w_ref, b_ref, g_ref, bt_ref, o_ref):
    x = x_ref[...]
    h = jnp.dot(x, w_ref[...], preferred_element_type=jnp.float32) + b_ref[...]
    h = jnp.maximum(h, 0.0)
    m = jnp.mean(h, axis=1, keepdims=True)
    c = h - m
    v = jnp.mean(c * c, axis=1, keepdims=True)
    o_ref[...] = c * lax.rsqrt(v + EPS) * g_ref[...] + bt_ref[...]


def _dense(rows, W, b, gamma, beta, BB=2048):
    B, D = rows.shape
    H = W.shape[1]
    return pl.pallas_call(
        _dense_body,
        grid=(B // BB,),
        in_specs=[
            pl.BlockSpec((BB, D), lambda i: (i, 0)),
            pl.BlockSpec((D, H), lambda i: (0, 0)),
            pl.BlockSpec((1, H), lambda i: (0, 0)),
            pl.BlockSpec((1, H), lambda i: (0, 0)),
            pl.BlockSpec((1, H), lambda i: (0, 0)),
        ],
        out_specs=pl.BlockSpec((BB, H), lambda i: (i, 0)),
        out_shape=jax.ShapeDtypeStruct((B, H), jnp.float32),
    )(rows, W, b.reshape(1, H), gamma.reshape(1, H), beta.reshape(1, H))


def kernel(user_input, table, W, b, gamma, beta):
    B = user_input.shape[0]
    V, D = table.shape
    info = plsc.get_sparse_core_info()
    NC, NS = info.num_cores, info.num_subcores
    NW = NC * NS
    idx = user_input.astype(jnp.int32).reshape(NW, B // NW)
    table_rm = _transpose(table.T)
    rows = _make_sc_gather(V, D, B, NC, NW)(table_rm, idx)
    return _dense(rows, W, b, gamma, beta)
